# baseline (speedup 1.0000x reference)
import functools

import jax
import jax.numpy as jnp
from jax import lax
from jax.experimental import pallas as pl
from jax.experimental.pallas import tpu as pltpu
from jax.experimental.pallas import tpu_sc as plsc

ALPHA = 0.2
NEG_SLOPE = 0.01

NC = 2
NS = 16
LANES = 16

NP = 100352
ROWS_PER_TEC = NP // 16
DEAD_NODE = 100000

W = 128
EROWS_P = 12544
E_PAD = EROWS_P * W
K = 8
KH = 4
SEG_ITERS = EROWS_P // NS // K
DEG_ITERS = EROWS_P // (NC * NS) // K


def _sc_mesh():
    return plsc.VectorSubcoreMesh(core_axis_name="c", subcore_axis_name="s",
                                  num_cores=NC, num_subcores=NS)


def _deg_body(col2d, zeros1d, out, dacc, ones, cbuf):
    cid = lax.axis_index("c")
    sid = lax.axis_index("s")
    wid = sid * NC + cid
    pltpu.sync_copy(zeros1d, dacc.at[pl.ds(sid * ROWS_PER_TEC, ROWS_PER_TEC)])
    for j in range(W // LANES):
        ones[pl.ds(j * LANES, LANES)] = jnp.ones((LANES,), jnp.float32)
    plsc.subcore_barrier()

    def body(i, _):
        wbase = wid * (DEG_ITERS * K) + i * K
        pltpu.sync_copy(col2d.at[pl.ds(wbase, K)], cbuf)
        for k in range(K):
            pltpu.sync_copy(ones, dacc.at[cbuf.at[k]], add=True)
        return 0

    lax.fori_loop(0, DEG_ITERS, body, 0)
    plsc.subcore_barrier()
    pltpu.sync_copy(dacc.at[pl.ds(sid * ROWS_PER_TEC, ROWS_PER_TEC)],
                    out.at[cid, 0, pl.ds(sid * ROWS_PER_TEC, ROWS_PER_TEC)])


_sc_deg = functools.partial(
    pl.kernel,
    out_type=jax.ShapeDtypeStruct((NC, 1, NP), jnp.float32),
    mesh=_sc_mesh(),
    compiler_params=pltpu.CompilerParams(use_tc_tiling_on_sc=False),
    scratch_types=[
        pltpu.VMEM_SHARED((NP,), jnp.float32),
        pltpu.VMEM((W,), jnp.float32),
        pltpu.VMEM((K, W), jnp.int32),
    ],
)(_deg_body)


def _segsum_body(yflat, row2d, col2d, zeros2d, agg,
                 acc, rowbuf, colbuf, row8a, row8b, stagea, stageb,
                 gsem, ssem):
    cid = lax.axis_index("c")
    sid = lax.axis_index("s")
    for sl in range(4):
        sidx = cid * 4 + sl
        pltpu.sync_copy(zeros2d, acc.at[pl.ds(sid * ROWS_PER_TEC, ROWS_PER_TEC)])
        plsc.subcore_barrier()

        tbase = sid * (SEG_ITERS * K)

        def load_idx(g):
            pltpu.sync_copy(row2d.at[pl.ds(tbase + g * K, K)], rowbuf)
            pltpu.sync_copy(col2d.at[pl.ds(tbase + g * K, K)], colbuf)

        def comp_row8(r8, off):
            for k in range(KH):
                for j in range(W // LANES):
                    v = rowbuf[off + k, pl.ds(j * LANES, LANES)]
                    r8[k, pl.ds(j * LANES, LANES)] = v * 8 + sidx

        def fire_g(r8, stg):
            return [pltpu.async_copy(yflat.at[r8.at[k]], stg.at[k], gsem)
                    for k in range(KH)]

        def fire_s(stg, off):
            return [pltpu.async_copy(stg.at[k], acc.at[colbuf.at[off + k]],
                                     ssem, add=True)
                    for k in range(KH)]

        def half(ga_cps, stg_in, off_in, r8_next, off_next, stg_next):
            for cp in ga_cps:
                cp.wait()
            scps = fire_s(stg_in, off_in)
            comp_row8(r8_next, off_next)
            return scps, fire_g(r8_next, stg_next)

        load_idx(0)
        comp_row8(row8a, 0)
        ga = fire_g(row8a, stagea)

        def body(i, ga_wait):
            ga = [pltpu.make_async_copy(yflat.at[row8a.at[k]], stagea.at[k],
                                        gsem) for k in range(KH)]
            sa, gb = half(ga, stagea, 0, row8b, KH, stageb)
            sb = fire_s_wait(gb, stageb)
            for cp in sa + sb:
                cp.wait()
            load_idx(i + 1)
            comp_row8(row8a, 0)
            fire_g(row8a, stagea)
            return 0

        def fire_s_wait(g_cps, stg):
            for cp in g_cps:
                cp.wait()
            return fire_s(stg, KH)

        lax.fori_loop(0, SEG_ITERS - 1, body, 0)
        ga = [pltpu.make_async_copy(yflat.at[row8a.at[k]], stagea.at[k], gsem)
              for k in range(KH)]
        sa, gb = half(ga, stagea, 0, row8b, KH, stageb)
        sb = fire_s_wait(gb, stageb)
        for cp in sa + sb:
            cp.wait()
        plsc.subcore_barrier()
        rbase = sid * ROWS_PER_TEC
        pltpu.sync_copy(acc.at[pl.ds(rbase, ROWS_PER_TEC)],
                        agg.at[sidx, pl.ds(rbase, ROWS_PER_TEC)])
        plsc.subcore_barrier()


_sc_segsum = functools.partial(
    pl.kernel,
    out_type=jax.ShapeDtypeStruct((8, NP, 16), jnp.float32),
    mesh=_sc_mesh(),
    compiler_params=pltpu.CompilerParams(use_tc_tiling_on_sc=False),
    scratch_types=[
        pltpu.VMEM_SHARED((NP, 16), jnp.float32),
        pltpu.VMEM((K, W), jnp.int32),
        pltpu.VMEM((K, W), jnp.int32),
        pltpu.VMEM((KH, W), jnp.int32),
        pltpu.VMEM((KH, W), jnp.int32),
        pltpu.VMEM((KH, W, 16), jnp.float32),
        pltpu.VMEM((KH, W, 16), jnp.float32),
        pltpu.SemaphoreType.DMA,
        pltpu.SemaphoreType.DMA,
    ],
)(_segsum_body)


_BN = 1024
_GRID = NP // _BN


def _dinv(dp):
    deg = dp[0] + dp[1]
    return jnp.where(deg > 0, lax.rsqrt(jnp.maximum(deg, 1e-12)), 0.0)


def _prelude_body(f_ref, wi_ref, bi_ref, dp_ref, x_ref, y_ref):
    xb = jnp.dot(f_ref[...], wi_ref[...], preferred_element_type=jnp.float32)
    xb = xb + bi_ref[...]
    xb = jnp.where(xb >= 0, xb, NEG_SLOPE * xb)
    x_ref[...] = xb
    y_ref[...] = xb * _dinv(dp_ref[...])[:, None]


def _tc_prelude(feature_p, W_in, b_in, degp):
    return pl.pallas_call(
        _prelude_body,
        grid=(_GRID,),
        in_specs=[
            pl.BlockSpec((_BN, 16), lambda i: (i, 0)),
            pl.BlockSpec((16, 128), lambda i: (0, 0)),
            pl.BlockSpec((1, 128), lambda i: (0, 0)),
            pl.BlockSpec((2, _BN), lambda i: (0, i)),
        ],
        out_specs=[
            pl.BlockSpec((_BN, 128), lambda i: (i, 0)),
            pl.BlockSpec((_BN, 128), lambda i: (i, 0)),
        ],
        out_shape=[
            jax.ShapeDtypeStruct((NP, 128), jnp.float32),
            jax.ShapeDtypeStruct((NP, 128), jnp.float32),
        ],
    )(feature_p, W_in, b_in.reshape(1, 128), degp)


def _mix(agg_ref, x_ref, dp_ref):
    dinv = _dinv(dp_ref[...])
    cat = jnp.concatenate([agg_ref[s] for s in range(8)], axis=-1)
    return (1.0 - ALPHA) * (cat * dinv[:, None]) + ALPHA * x_ref[...]


def _layer_body(agg_ref, x_ref, w_ref, dp_ref, y_ref):
    dinv = _dinv(dp_ref[...])
    h = jnp.dot(_mix(agg_ref, x_ref, dp_ref), w_ref[...],
                preferred_element_type=jnp.float32)
    y_ref[...] = h * dinv[:, None]


def _tc_layer(agg, x, W1, degp):
    return pl.pallas_call(
        _layer_body,
        grid=(_GRID,),
        in_specs=[
            pl.BlockSpec((8, _BN, 16), lambda i: (0, i, 0)),
            pl.BlockSpec((_BN, 128), lambda i: (i, 0)),
            pl.BlockSpec((128, 128), lambda i: (0, 0)),
            pl.BlockSpec((2, _BN), lambda i: (0, i)),
        ],
        out_specs=pl.BlockSpec((_BN, 128), lambda i: (i, 0)),
        out_shape=jax.ShapeDtypeStruct((NP, 128), jnp.float32),
    )(agg, x, W1, degp)


def _final_body(agg_ref, x_ref, w_ref, wo_ref, bo_ref, dp_ref, o_ref):
    h = jnp.dot(_mix(agg_ref, x_ref, dp_ref), w_ref[...],
                preferred_element_type=jnp.float32)
    o_ref[...] = jnp.dot(h, wo_ref[...], preferred_element_type=jnp.float32) \
        + bo_ref[...]


def _tc_final(agg, x, W2, W_out_p, b_out_p, degp):
    return pl.pallas_call(
        _final_body,
        grid=(_GRID,),
        in_specs=[
            pl.BlockSpec((8, _BN, 16), lambda i: (0, i, 0)),
            pl.BlockSpec((_BN, 128), lambda i: (i, 0)),
            pl.BlockSpec((128, 128), lambda i: (0, 0)),
            pl.BlockSpec((128, 8), lambda i: (0, 0)),
            pl.BlockSpec((1, 8), lambda i: (0, 0)),
            pl.BlockSpec((2, _BN), lambda i: (0, i)),
        ],
        out_specs=pl.BlockSpec((_BN, 8), lambda i: (i, 0)),
        out_shape=jax.ShapeDtypeStruct((NP, 8), jnp.float32),
    )(agg, x, W2, W_out_p, b_out_p, degp)


def kernel(feature, edge_index, edge_type, W_in, b_in, W1, W2, W_out, b_out):
    n, _ = feature.shape
    e = edge_index.shape[1]
    pad_i = jnp.arange(E_PAD - e, dtype=jnp.int32)
    row2d = jnp.concatenate(
        [edge_index[0], pad_i % jnp.int32(n)]).reshape(EROWS_P, W)
    col2d = jnp.concatenate(
        [edge_index[1], DEAD_NODE + pad_i % jnp.int32(NP - DEAD_NODE)]
    ).reshape(EROWS_P, W)
    zeros1d = jnp.zeros((ROWS_PER_TEC,), jnp.float32)
    zeros2d = jnp.zeros((ROWS_PER_TEC, 16), jnp.float32)
    feature_p = jnp.pad(feature, ((0, NP - n), (0, 0)))
    W_out_p = jnp.pad(W_out, ((0, 0), (0, 8 - W_out.shape[1])))
    b_out_p = jnp.pad(b_out, (0, 8 - b_out.shape[0])).reshape(1, 8)

    degp = _sc_deg(col2d, zeros1d).reshape(NC, NP)
    x, y1 = _tc_prelude(feature_p, W_in, b_in, degp)
    agg1 = _sc_segsum(y1.reshape(NP * 8, 16), row2d, col2d, zeros2d)
    y2 = _tc_layer(agg1, x, W1, degp)
    agg2 = _sc_segsum(y2.reshape(NP * 8, 16), row2d, col2d, zeros2d)
    out = _tc_final(agg2, x, W2, W_out_p, b_out_p, degp)
    return out[:n, :3]


# speedup vs baseline: 1.1862x; 1.1862x over previous
import functools

import jax
import jax.numpy as jnp
from jax import lax
from jax.experimental import pallas as pl
from jax.experimental.pallas import tpu as pltpu
from jax.experimental.pallas import tpu_sc as plsc

ALPHA = 0.2
NEG_SLOPE = 0.01

NC = 2
NS = 16
LANES = 16

NP = 100352
ROWS_PER_TEC = NP // 16
DEAD_NODE = 100000

W = 128
EROWS_P = 12544
E_PAD = EROWS_P * W
K = 8
KH = 4
SB = 16
NH = SB // KH
SEG_ITERS = EROWS_P // NS // K
SEG_SBS = EROWS_P // NS // SB
DEG_ITERS = EROWS_P // (NC * NS) // K


def _sc_mesh():
    return plsc.VectorSubcoreMesh(core_axis_name="c", subcore_axis_name="s",
                                  num_cores=NC, num_subcores=NS)


def _deg_body(col2d, zeros1d, out, dacc, ones, cbuf):
    cid = lax.axis_index("c")
    sid = lax.axis_index("s")
    wid = sid * NC + cid
    pltpu.sync_copy(zeros1d, dacc.at[pl.ds(sid * ROWS_PER_TEC, ROWS_PER_TEC)])
    for j in range(W // LANES):
        ones[pl.ds(j * LANES, LANES)] = jnp.ones((LANES,), jnp.float32)
    plsc.subcore_barrier()

    def body(i, _):
        wbase = wid * (DEG_ITERS * K) + i * K
        pltpu.sync_copy(col2d.at[pl.ds(wbase, K)], cbuf)
        for k in range(K):
            pltpu.sync_copy(ones, dacc.at[cbuf.at[k]], add=True)
        return 0

    lax.fori_loop(0, DEG_ITERS, body, 0)
    plsc.subcore_barrier()
    pltpu.sync_copy(dacc.at[pl.ds(sid * ROWS_PER_TEC, ROWS_PER_TEC)],
                    out.at[cid, 0, pl.ds(sid * ROWS_PER_TEC, ROWS_PER_TEC)])


_sc_deg = functools.partial(
    pl.kernel,
    out_type=jax.ShapeDtypeStruct((NC, 1, NP), jnp.float32),
    mesh=_sc_mesh(),
    compiler_params=pltpu.CompilerParams(use_tc_tiling_on_sc=False),
    scratch_types=[
        pltpu.VMEM_SHARED((NP,), jnp.float32),
        pltpu.VMEM((W,), jnp.float32),
        pltpu.VMEM((K, W), jnp.int32),
    ],
)(_deg_body)


def _segsum_body(yflat, row2d, col2d, zeros2d, agg,
                 acc, rowbuf, colbuf, row8a, row8b, stagea, stageb,
                 gsem, ssem, isem):
    cid = lax.axis_index("c")
    sid = lax.axis_index("s")
    for sl in range(4):
        sidx = cid * 4 + sl
        pltpu.sync_copy(zeros2d, acc.at[pl.ds(sid * ROWS_PER_TEC, ROWS_PER_TEC)])
        plsc.subcore_barrier()

        tbase = sid * (SEG_ITERS * K)

        def load_idx(g):
            c1 = pltpu.async_copy(row2d.at[pl.ds(tbase + g * SB, SB)],
                                  rowbuf, isem)
            c2 = pltpu.async_copy(col2d.at[pl.ds(tbase + g * SB, SB)],
                                  colbuf, isem)
            return [c1, c2]

        def comp_row8(r8, off):
            for k in range(KH):
                for j in range(W // LANES):
                    v = rowbuf[off + k, pl.ds(j * LANES, LANES)]
                    r8[k, pl.ds(j * LANES, LANES)] = v * 8 + sidx

        def fire_g(r8, stg):
            return [pltpu.async_copy(yflat.at[r8.at[k]], stg.at[k], gsem)
                    for k in range(KH)]

        def fire_s(stg, off):
            return [pltpu.async_copy(stg.at[k], acc.at[colbuf.at[off + k]],
                                     ssem, add=True)
                    for k in range(KH)]

        def wait(cps):
            for cp in cps:
                cp.wait()

        stages = [stagea, stageb]
        r8s = [row8a, row8b]

        def superblock(i, last):
            ga = [pltpu.make_async_copy(yflat.at[row8a.at[k]],
                                        stagea.at[k], gsem)
                  for k in range(KH)]
            sprev = []
            for h in range(NH):
                wait(ga)
                s_h = fire_s(stages[h % 2], h * KH)
                if h + 1 < NH:
                    comp_row8(r8s[(h + 1) % 2], (h + 1) * KH)
                    wait(sprev)
                    ga = fire_g(r8s[(h + 1) % 2], stages[(h + 1) % 2])
                    sprev = s_h
                else:
                    wait(sprev)
                    wait(s_h)
            if not last:
                wait(load_idx(i + 1))
                comp_row8(row8a, 0)
                fire_g(row8a, stagea)

        wait(load_idx(0))
        comp_row8(row8a, 0)
        fire_g(row8a, stagea)

        def body(i, _):
            superblock(i, False)
            return 0

        lax.fori_loop(0, SEG_SBS - 1, body, 0)
        superblock(SEG_SBS - 1, True)
        plsc.subcore_barrier()
        rbase = sid * ROWS_PER_TEC
        pltpu.sync_copy(acc.at[pl.ds(rbase, ROWS_PER_TEC)],
                        agg.at[sidx, pl.ds(rbase, ROWS_PER_TEC)])
        plsc.subcore_barrier()


_sc_segsum = functools.partial(
    pl.kernel,
    out_type=jax.ShapeDtypeStruct((8, NP, 16), jnp.float32),
    mesh=_sc_mesh(),
    compiler_params=pltpu.CompilerParams(use_tc_tiling_on_sc=False),
    scratch_types=[
        pltpu.VMEM_SHARED((NP, 16), jnp.float32),
        pltpu.VMEM((SB, W), jnp.int32),
        pltpu.VMEM((SB, W), jnp.int32),
        pltpu.VMEM((KH, W), jnp.int32),
        pltpu.VMEM((KH, W), jnp.int32),
        pltpu.VMEM((KH, W, 16), jnp.float32),
        pltpu.VMEM((KH, W, 16), jnp.float32),
        pltpu.SemaphoreType.DMA,
        pltpu.SemaphoreType.DMA,
        pltpu.SemaphoreType.DMA,
    ],
)(_segsum_body)


_BN = 1024
_GRID = NP // _BN


def _dinv(dp):
    deg = dp[0] + dp[1]
    return jnp.where(deg > 0, lax.rsqrt(jnp.maximum(deg, 1e-12)), 0.0)


def _prelude_body(f_ref, wi_ref, bi_ref, dp_ref, x_ref, y_ref):
    xb = jnp.dot(f_ref[...], wi_ref[...], preferred_element_type=jnp.float32)
    xb = xb + bi_ref[...]
    xb = jnp.where(xb >= 0, xb, NEG_SLOPE * xb)
    x_ref[...] = xb
    y_ref[...] = xb * _dinv(dp_ref[...])[:, None]


def _tc_prelude(feature_p, W_in, b_in, degp):
    return pl.pallas_call(
        _prelude_body,
        grid=(_GRID,),
        in_specs=[
            pl.BlockSpec((_BN, 16), lambda i: (i, 0)),
            pl.BlockSpec((16, 128), lambda i: (0, 0)),
            pl.BlockSpec((1, 128), lambda i: (0, 0)),
            pl.BlockSpec((2, _BN), lambda i: (0, i)),
        ],
        out_specs=[
            pl.BlockSpec((_BN, 128), lambda i: (i, 0)),
            pl.BlockSpec((_BN, 128), lambda i: (i, 0)),
        ],
        out_shape=[
            jax.ShapeDtypeStruct((NP, 128), jnp.float32),
            jax.ShapeDtypeStruct((NP, 128), jnp.float32),
        ],
    )(feature_p, W_in, b_in.reshape(1, 128), degp)


def _mix(agg_ref, x_ref, dp_ref):
    dinv = _dinv(dp_ref[...])
    cat = jnp.concatenate([agg_ref[s] for s in range(8)], axis=-1)
    return (1.0 - ALPHA) * (cat * dinv[:, None]) + ALPHA * x_ref[...]


def _layer_body(agg_ref, x_ref, w_ref, dp_ref, y_ref):
    dinv = _dinv(dp_ref[...])
    h = jnp.dot(_mix(agg_ref, x_ref, dp_ref), w_ref[...],
                preferred_element_type=jnp.float32)
    y_ref[...] = h * dinv[:, None]


def _tc_layer(agg, x, W1, degp):
    return pl.pallas_call(
        _layer_body,
        grid=(_GRID,),
        in_specs=[
            pl.BlockSpec((8, _BN, 16), lambda i: (0, i, 0)),
            pl.BlockSpec((_BN, 128), lambda i: (i, 0)),
            pl.BlockSpec((128, 128), lambda i: (0, 0)),
            pl.BlockSpec((2, _BN), lambda i: (0, i)),
        ],
        out_specs=pl.BlockSpec((_BN, 128), lambda i: (i, 0)),
        out_shape=jax.ShapeDtypeStruct((NP, 128), jnp.float32),
    )(agg, x, W1, degp)


def _final_body(agg_ref, x_ref, w_ref, wo_ref, bo_ref, dp_ref, o_ref):
    h = jnp.dot(_mix(agg_ref, x_ref, dp_ref), w_ref[...],
                preferred_element_type=jnp.float32)
    o_ref[...] = jnp.dot(h, wo_ref[...], preferred_element_type=jnp.float32) \
        + bo_ref[...]


def _tc_final(agg, x, W2, W_out_p, b_out_p, degp):
    return pl.pallas_call(
        _final_body,
        grid=(_GRID,),
        in_specs=[
            pl.BlockSpec((8, _BN, 16), lambda i: (0, i, 0)),
            pl.BlockSpec((_BN, 128), lambda i: (i, 0)),
            pl.BlockSpec((128, 128), lambda i: (0, 0)),
            pl.BlockSpec((128, 8), lambda i: (0, 0)),
            pl.BlockSpec((1, 8), lambda i: (0, 0)),
            pl.BlockSpec((2, _BN), lambda i: (0, i)),
        ],
        out_specs=pl.BlockSpec((_BN, 8), lambda i: (i, 0)),
        out_shape=jax.ShapeDtypeStruct((NP, 8), jnp.float32),
    )(agg, x, W2, W_out_p, b_out_p, degp)


def kernel(feature, edge_index, edge_type, W_in, b_in, W1, W2, W_out, b_out):
    n, _ = feature.shape
    e = edge_index.shape[1]
    pad_i = jnp.arange(E_PAD - e, dtype=jnp.int32)
    row2d = jnp.concatenate(
        [edge_index[0], pad_i % jnp.int32(n)]).reshape(EROWS_P, W)
    col2d = jnp.concatenate(
        [edge_index[1], DEAD_NODE + pad_i % jnp.int32(NP - DEAD_NODE)]
    ).reshape(EROWS_P, W)
    zeros1d = jnp.zeros((ROWS_PER_TEC,), jnp.float32)
    zeros2d = jnp.zeros((ROWS_PER_TEC, 16), jnp.float32)
    feature_p = jnp.pad(feature, ((0, NP - n), (0, 0)))
    W_out_p = jnp.pad(W_out, ((0, 0), (0, 8 - W_out.shape[1])))
    b_out_p = jnp.pad(b_out, (0, 8 - b_out.shape[0])).reshape(1, 8)

    degp = _sc_deg(col2d, zeros1d).reshape(NC, NP)
    x, y1 = _tc_prelude(feature_p, W_in, b_in, degp)
    agg1 = _sc_segsum(y1.reshape(NP * 8, 16), row2d, col2d, zeros2d)
    y2 = _tc_layer(agg1, x, W1, degp)
    agg2 = _sc_segsum(y2.reshape(NP * 8, 16), row2d, col2d, zeros2d)
    out = _tc_final(agg2, x, W2, W_out_p, b_out_p, degp)
    return out[:n, :3]


# speedup vs baseline: 1.3326x; 1.1234x over previous
import functools

import jax
import jax.numpy as jnp
from jax import lax
from jax.experimental import pallas as pl
from jax.experimental.pallas import tpu as pltpu
from jax.experimental.pallas import tpu_sc as plsc

ALPHA = 0.2
NEG_SLOPE = 0.01

NC = 2
NS = 16
LANES = 16

NP = 100352
ROWS_PER_TEC = NP // 16
DEAD_NODE = 100000

W = 128
EROWS_P = 12544
E_PAD = EROWS_P * W
K = 8
KH = 2
SB = 16
NH = SB // KH
DEPTH = 4
SEG_ITERS = EROWS_P // NS // K
SEG_SBS = EROWS_P // NS // SB
DEG_ITERS = EROWS_P // (NC * NS) // K


def _sc_mesh():
    return plsc.VectorSubcoreMesh(core_axis_name="c", subcore_axis_name="s",
                                  num_cores=NC, num_subcores=NS)


def _deg_body(col2d, zeros1d, out, dacc, ones, cbuf):
    cid = lax.axis_index("c")
    sid = lax.axis_index("s")
    wid = sid * NC + cid
    pltpu.sync_copy(zeros1d, dacc.at[pl.ds(sid * ROWS_PER_TEC, ROWS_PER_TEC)])
    for j in range(W // LANES):
        ones[pl.ds(j * LANES, LANES)] = jnp.ones((LANES,), jnp.float32)
    plsc.subcore_barrier()

    def body(i, _):
        wbase = wid * (DEG_ITERS * K) + i * K
        pltpu.sync_copy(col2d.at[pl.ds(wbase, K)], cbuf)
        for k in range(K):
            pltpu.sync_copy(ones, dacc.at[cbuf.at[k]], add=True)
        return 0

    lax.fori_loop(0, DEG_ITERS, body, 0)
    plsc.subcore_barrier()
    pltpu.sync_copy(dacc.at[pl.ds(sid * ROWS_PER_TEC, ROWS_PER_TEC)],
                    out.at[cid, 0, pl.ds(sid * ROWS_PER_TEC, ROWS_PER_TEC)])


_sc_deg = functools.partial(
    pl.kernel,
    out_type=jax.ShapeDtypeStruct((NC, 1, NP), jnp.float32),
    mesh=_sc_mesh(),
    compiler_params=pltpu.CompilerParams(use_tc_tiling_on_sc=False),
    scratch_types=[
        pltpu.VMEM_SHARED((NP,), jnp.float32),
        pltpu.VMEM((W,), jnp.float32),
        pltpu.VMEM((K, W), jnp.int32),
    ],
)(_deg_body)


def _segsum_body(yflat, row2d, col2d, zeros2d, agg,
                 acc, rowbuf, colbuf, r80, r81, r82, r83,
                 st0, st1, st2, st3, gsem, ssem, isem):
    cid = lax.axis_index("c")
    sid = lax.axis_index("s")
    r8s = [r80, r81, r82, r83]
    stages = [st0, st1, st2, st3]
    for sl in range(4):
        sidx = cid * 4 + sl
        pltpu.sync_copy(zeros2d, acc.at[pl.ds(sid * ROWS_PER_TEC, ROWS_PER_TEC)])
        plsc.subcore_barrier()

        tbase = sid * (SEG_ITERS * K)

        def load_idx(g):
            c1 = pltpu.async_copy(row2d.at[pl.ds(tbase + g * SB, SB)],
                                  rowbuf, isem)
            c2 = pltpu.async_copy(col2d.at[pl.ds(tbase + g * SB, SB)],
                                  colbuf, isem)
            return [c1, c2]

        def comp_row8(r8, off):
            for k in range(KH):
                for j in range(W // LANES):
                    v = rowbuf[off + k, pl.ds(j * LANES, LANES)]
                    r8[k, pl.ds(j * LANES, LANES)] = v * 8 + sidx

        def fire_g(r):
            return [pltpu.async_copy(yflat.at[r8s[r].at[k]],
                                     stages[r].at[k], gsem)
                    for k in range(KH)]

        def remake_g(r):
            return [pltpu.make_async_copy(yflat.at[r8s[r].at[k]],
                                          stages[r].at[k], gsem)
                    for k in range(KH)]

        def fire_s(r, off):
            return [pltpu.async_copy(stages[r].at[k],
                                     acc.at[colbuf.at[off + k]],
                                     ssem, add=True)
                    for k in range(KH)]

        def wait(cps):
            for cp in cps:
                cp.wait()

        def prime(g):
            wait(load_idx(g))
            for r in range(DEPTH - 1):
                comp_row8(r8s[r], r * KH)
                fire_g(r)

        def superblock(i, last):
            ga = {h: remake_g(h) for h in range(DEPTH - 1)}
            sc = {}
            for h in range(NH):
                wait(ga.pop(h))
                sc[h] = fire_s(h % DEPTH, h * KH)
                if h + DEPTH - 1 < NH:
                    nxt = h + DEPTH - 1
                    comp_row8(r8s[nxt % DEPTH], nxt * KH)
                    if h >= 1:
                        wait(sc.pop(h - 1))
                    ga[nxt] = fire_g(nxt % DEPTH)
            for h in sorted(sc):
                wait(sc[h])
            if not last:
                prime(i + 1)

        prime(0)

        def body(i, _):
            superblock(i, False)
            return 0

        lax.fori_loop(0, SEG_SBS - 1, body, 0)
        superblock(SEG_SBS - 1, True)
        plsc.subcore_barrier()
        rbase = sid * ROWS_PER_TEC
        pltpu.sync_copy(acc.at[pl.ds(rbase, ROWS_PER_TEC)],
                        agg.at[sidx, pl.ds(rbase, ROWS_PER_TEC)])
        plsc.subcore_barrier()


_sc_segsum = functools.partial(
    pl.kernel,
    out_type=jax.ShapeDtypeStruct((8, NP, 16), jnp.float32),
    mesh=_sc_mesh(),
    compiler_params=pltpu.CompilerParams(use_tc_tiling_on_sc=False),
    scratch_types=[
        pltpu.VMEM_SHARED((NP, 16), jnp.float32),
        pltpu.VMEM((SB, W), jnp.int32),
        pltpu.VMEM((SB, W), jnp.int32),
        pltpu.VMEM((KH, W), jnp.int32),
        pltpu.VMEM((KH, W), jnp.int32),
        pltpu.VMEM((KH, W), jnp.int32),
        pltpu.VMEM((KH, W), jnp.int32),
        pltpu.VMEM((KH, W, 16), jnp.float32),
        pltpu.VMEM((KH, W, 16), jnp.float32),
        pltpu.VMEM((KH, W, 16), jnp.float32),
        pltpu.VMEM((KH, W, 16), jnp.float32),
        pltpu.SemaphoreType.DMA,
        pltpu.SemaphoreType.DMA,
        pltpu.SemaphoreType.DMA,
    ],
)(_segsum_body)


_BN = 1024
_GRID = NP // _BN


def _dinv(dp):
    deg = dp[0] + dp[1]
    return jnp.where(deg > 0, lax.rsqrt(jnp.maximum(deg, 1e-12)), 0.0)


def _prelude_body(f_ref, wi_ref, bi_ref, dp_ref, x_ref, y_ref):
    xb = jnp.dot(f_ref[...], wi_ref[...], preferred_element_type=jnp.float32)
    xb = xb + bi_ref[...]
    xb = jnp.where(xb >= 0, xb, NEG_SLOPE * xb)
    x_ref[...] = xb
    y_ref[...] = xb * _dinv(dp_ref[...])[:, None]


def _tc_prelude(feature_p, W_in, b_in, degp):
    return pl.pallas_call(
        _prelude_body,
        grid=(_GRID,),
        in_specs=[
            pl.BlockSpec((_BN, 16), lambda i: (i, 0)),
            pl.BlockSpec((16, 128), lambda i: (0, 0)),
            pl.BlockSpec((1, 128), lambda i: (0, 0)),
            pl.BlockSpec((2, _BN), lambda i: (0, i)),
        ],
        out_specs=[
            pl.BlockSpec((_BN, 128), lambda i: (i, 0)),
            pl.BlockSpec((_BN, 128), lambda i: (i, 0)),
        ],
        out_shape=[
            jax.ShapeDtypeStruct((NP, 128), jnp.float32),
            jax.ShapeDtypeStruct((NP, 128), jnp.float32),
        ],
    )(feature_p, W_in, b_in.reshape(1, 128), degp)


def _mix(agg_ref, x_ref, dp_ref):
    dinv = _dinv(dp_ref[...])
    cat = jnp.concatenate([agg_ref[s] for s in range(8)], axis=-1)
    return (1.0 - ALPHA) * (cat * dinv[:, None]) + ALPHA * x_ref[...]


def _layer_body(agg_ref, x_ref, w_ref, dp_ref, y_ref):
    dinv = _dinv(dp_ref[...])
    h = jnp.dot(_mix(agg_ref, x_ref, dp_ref), w_ref[...],
                preferred_element_type=jnp.float32)
    y_ref[...] = h * dinv[:, None]


def _tc_layer(agg, x, W1, degp):
    return pl.pallas_call(
        _layer_body,
        grid=(_GRID,),
        in_specs=[
            pl.BlockSpec((8, _BN, 16), lambda i: (0, i, 0)),
            pl.BlockSpec((_BN, 128), lambda i: (i, 0)),
            pl.BlockSpec((128, 128), lambda i: (0, 0)),
            pl.BlockSpec((2, _BN), lambda i: (0, i)),
        ],
        out_specs=pl.BlockSpec((_BN, 128), lambda i: (i, 0)),
        out_shape=jax.ShapeDtypeStruct((NP, 128), jnp.float32),
    )(agg, x, W1, degp)


def _final_body(agg_ref, x_ref, w_ref, wo_ref, bo_ref, dp_ref, o_ref):
    h = jnp.dot(_mix(agg_ref, x_ref, dp_ref), w_ref[...],
                preferred_element_type=jnp.float32)
    o_ref[...] = jnp.dot(h, wo_ref[...], preferred_element_type=jnp.float32) \
        + bo_ref[...]


def _tc_final(agg, x, W2, W_out_p, b_out_p, degp):
    return pl.pallas_call(
        _final_body,
        grid=(_GRID,),
        in_specs=[
            pl.BlockSpec((8, _BN, 16), lambda i: (0, i, 0)),
            pl.BlockSpec((_BN, 128), lambda i: (i, 0)),
            pl.BlockSpec((128, 128), lambda i: (0, 0)),
            pl.BlockSpec((128, 8), lambda i: (0, 0)),
            pl.BlockSpec((1, 8), lambda i: (0, 0)),
            pl.BlockSpec((2, _BN), lambda i: (0, i)),
        ],
        out_specs=pl.BlockSpec((_BN, 8), lambda i: (i, 0)),
        out_shape=jax.ShapeDtypeStruct((NP, 8), jnp.float32),
    )(agg, x, W2, W_out_p, b_out_p, degp)


def kernel(feature, edge_index, edge_type, W_in, b_in, W1, W2, W_out, b_out):
    n, _ = feature.shape
    e = edge_index.shape[1]
    pad_i = jnp.arange(E_PAD - e, dtype=jnp.int32)
    row2d = jnp.concatenate(
        [edge_index[0], pad_i % jnp.int32(n)]).reshape(EROWS_P, W)
    col2d = jnp.concatenate(
        [edge_index[1], DEAD_NODE + pad_i % jnp.int32(NP - DEAD_NODE)]
    ).reshape(EROWS_P, W)
    zeros1d = jnp.zeros((ROWS_PER_TEC,), jnp.float32)
    zeros2d = jnp.zeros((ROWS_PER_TEC, 16), jnp.float32)
    feature_p = jnp.pad(feature, ((0, NP - n), (0, 0)))
    W_out_p = jnp.pad(W_out, ((0, 0), (0, 8 - W_out.shape[1])))
    b_out_p = jnp.pad(b_out, (0, 8 - b_out.shape[0])).reshape(1, 8)

    degp = _sc_deg(col2d, zeros1d).reshape(NC, NP)
    x, y1 = _tc_prelude(feature_p, W_in, b_in, degp)
    agg1 = _sc_segsum(y1.reshape(NP * 8, 16), row2d, col2d, zeros2d)
    y2 = _tc_layer(agg1, x, W1, degp)
    agg2 = _sc_segsum(y2.reshape(NP * 8, 16), row2d, col2d, zeros2d)
    out = _tc_final(agg2, x, W2, W_out_p, b_out_p, degp)
    return out[:n, :3]


# speedup vs baseline: 1.4943x; 1.1213x over previous
import functools

import jax
import jax.numpy as jnp
from jax import lax
from jax.experimental import pallas as pl
from jax.experimental.pallas import tpu as pltpu
from jax.experimental.pallas import tpu_sc as plsc

ALPHA = 0.2
NEG_SLOPE = 0.01

NC = 2
NS = 16
LANES = 16

NP = 100352
ROWS_PER_TEC = NP // 16
DEAD_NODE = 100000

W = 128
EROWS_P = 12544
E_PAD = EROWS_P * W
K = 8
KH = 2
SB = 16
NH = SB // KH
DEPTH = 6
SEG_ITERS = EROWS_P // NS // K
SEG_SBS = EROWS_P // NS // SB
DEG_ITERS = EROWS_P // (NC * NS) // K


def _sc_mesh():
    return plsc.VectorSubcoreMesh(core_axis_name="c", subcore_axis_name="s",
                                  num_cores=NC, num_subcores=NS)


def _deg_body(col2d, zeros1d, out, dacc, ones, cbuf):
    cid = lax.axis_index("c")
    sid = lax.axis_index("s")
    wid = sid * NC + cid
    pltpu.sync_copy(zeros1d, dacc.at[pl.ds(sid * ROWS_PER_TEC, ROWS_PER_TEC)])
    for j in range(W // LANES):
        ones[pl.ds(j * LANES, LANES)] = jnp.ones((LANES,), jnp.float32)
    plsc.subcore_barrier()

    def body(i, _):
        wbase = wid * (DEG_ITERS * K) + i * K
        pltpu.sync_copy(col2d.at[pl.ds(wbase, K)], cbuf)
        for k in range(K):
            pltpu.sync_copy(ones, dacc.at[cbuf.at[k]], add=True)
        return 0

    lax.fori_loop(0, DEG_ITERS, body, 0)
    plsc.subcore_barrier()
    pltpu.sync_copy(dacc.at[pl.ds(sid * ROWS_PER_TEC, ROWS_PER_TEC)],
                    out.at[cid, 0, pl.ds(sid * ROWS_PER_TEC, ROWS_PER_TEC)])


_sc_deg = functools.partial(
    pl.kernel,
    out_type=jax.ShapeDtypeStruct((NC, 1, NP), jnp.float32),
    mesh=_sc_mesh(),
    compiler_params=pltpu.CompilerParams(use_tc_tiling_on_sc=False),
    scratch_types=[
        pltpu.VMEM_SHARED((NP,), jnp.float32),
        pltpu.VMEM((W,), jnp.float32),
        pltpu.VMEM((K, W), jnp.int32),
    ],
)(_deg_body)


def _segsum_body(yflat, row2d, col2d, zeros2d, agg,
                 acc, rowbuf, colbuf, r80, r81, r82, r83, r84, r85,
                 st0, st1, st2, st3, st4, st5, gsem, ssem, isem):
    cid = lax.axis_index("c")
    sid = lax.axis_index("s")
    r8s = [r80, r81, r82, r83, r84, r85]
    stages = [st0, st1, st2, st3, st4, st5]
    for sl in range(4):
        sidx = cid * 4 + sl
        pltpu.sync_copy(zeros2d, acc.at[pl.ds(sid * ROWS_PER_TEC, ROWS_PER_TEC)])
        plsc.subcore_barrier()

        tbase = sid * (SEG_ITERS * K)

        def load_idx(g):
            c1 = pltpu.async_copy(row2d.at[pl.ds(tbase + g * SB, SB)],
                                  rowbuf, isem)
            c2 = pltpu.async_copy(col2d.at[pl.ds(tbase + g * SB, SB)],
                                  colbuf, isem)
            return [c1, c2]

        def comp_row8(r8, off):
            for k in range(KH):
                for j in range(W // LANES):
                    v = rowbuf[off + k, pl.ds(j * LANES, LANES)]
                    r8[k, pl.ds(j * LANES, LANES)] = v * 8 + sidx

        def fire_g(r):
            return [pltpu.async_copy(yflat.at[r8s[r].at[k]],
                                     stages[r].at[k], gsem)
                    for k in range(KH)]

        def remake_g(r):
            return [pltpu.make_async_copy(yflat.at[r8s[r].at[k]],
                                          stages[r].at[k], gsem)
                    for k in range(KH)]

        def fire_s(r, off):
            return [pltpu.async_copy(stages[r].at[k],
                                     acc.at[colbuf.at[off + k]],
                                     ssem, add=True)
                    for k in range(KH)]

        def wait(cps):
            for cp in cps:
                cp.wait()

        def prime(g):
            wait(load_idx(g))
            for r in range(DEPTH - 1):
                comp_row8(r8s[r], r * KH)
                fire_g(r)

        def superblock(i, last):
            ga = {h: remake_g(h) for h in range(DEPTH - 1)}
            sc = {}
            for h in range(NH):
                wait(ga.pop(h))
                sc[h] = fire_s(h % DEPTH, h * KH)
                if h + DEPTH - 1 < NH:
                    nxt = h + DEPTH - 1
                    comp_row8(r8s[nxt % DEPTH], nxt * KH)
                    if h >= 1:
                        wait(sc.pop(h - 1))
                    ga[nxt] = fire_g(nxt % DEPTH)
            for h in sorted(sc):
                wait(sc[h])
            if not last:
                prime(i + 1)

        prime(0)

        def body(i, _):
            superblock(i, False)
            return 0

        lax.fori_loop(0, SEG_SBS - 1, body, 0)
        superblock(SEG_SBS - 1, True)
        plsc.subcore_barrier()
        rbase = sid * ROWS_PER_TEC
        pltpu.sync_copy(acc.at[pl.ds(rbase, ROWS_PER_TEC)],
                        agg.at[sidx, pl.ds(rbase, ROWS_PER_TEC)])
        plsc.subcore_barrier()


_sc_segsum = functools.partial(
    pl.kernel,
    out_type=jax.ShapeDtypeStruct((8, NP, 16), jnp.float32),
    mesh=_sc_mesh(),
    compiler_params=pltpu.CompilerParams(use_tc_tiling_on_sc=False),
    scratch_types=[
        pltpu.VMEM_SHARED((NP, 16), jnp.float32),
        pltpu.VMEM((SB, W), jnp.int32),
        pltpu.VMEM((SB, W), jnp.int32),
        pltpu.VMEM((KH, W), jnp.int32),
        pltpu.VMEM((KH, W), jnp.int32),
        pltpu.VMEM((KH, W), jnp.int32),
        pltpu.VMEM((KH, W), jnp.int32),
        pltpu.VMEM((KH, W), jnp.int32),
        pltpu.VMEM((KH, W), jnp.int32),
        pltpu.VMEM((KH, W, 16), jnp.float32),
        pltpu.VMEM((KH, W, 16), jnp.float32),
        pltpu.VMEM((KH, W, 16), jnp.float32),
        pltpu.VMEM((KH, W, 16), jnp.float32),
        pltpu.VMEM((KH, W, 16), jnp.float32),
        pltpu.VMEM((KH, W, 16), jnp.float32),
        pltpu.SemaphoreType.DMA,
        pltpu.SemaphoreType.DMA,
        pltpu.SemaphoreType.DMA,
    ],
)(_segsum_body)


_BN = 1024
_GRID = NP // _BN


def _dinv(dp):
    deg = dp[0] + dp[1]
    return jnp.where(deg > 0, lax.rsqrt(jnp.maximum(deg, 1e-12)), 0.0)


def _prelude_body(f_ref, wi_ref, bi_ref, dp_ref, x_ref, y_ref):
    xb = jnp.dot(f_ref[...], wi_ref[...], preferred_element_type=jnp.float32)
    xb = xb + bi_ref[...]
    xb = jnp.where(xb >= 0, xb, NEG_SLOPE * xb)
    x_ref[...] = xb
    y_ref[...] = xb * _dinv(dp_ref[...])[:, None]


def _tc_prelude(feature_p, W_in, b_in, degp):
    return pl.pallas_call(
        _prelude_body,
        grid=(_GRID,),
        in_specs=[
            pl.BlockSpec((_BN, 16), lambda i: (i, 0)),
            pl.BlockSpec((16, 128), lambda i: (0, 0)),
            pl.BlockSpec((1, 128), lambda i: (0, 0)),
            pl.BlockSpec((2, _BN), lambda i: (0, i)),
        ],
        out_specs=[
            pl.BlockSpec((_BN, 128), lambda i: (i, 0)),
            pl.BlockSpec((_BN, 128), lambda i: (i, 0)),
        ],
        out_shape=[
            jax.ShapeDtypeStruct((NP, 128), jnp.float32),
            jax.ShapeDtypeStruct((NP, 128), jnp.float32),
        ],
    )(feature_p, W_in, b_in.reshape(1, 128), degp)


def _mix(agg_ref, x_ref, dp_ref):
    dinv = _dinv(dp_ref[...])
    cat = jnp.concatenate([agg_ref[s] for s in range(8)], axis=-1)
    return (1.0 - ALPHA) * (cat * dinv[:, None]) + ALPHA * x_ref[...]


def _layer_body(agg_ref, x_ref, w_ref, dp_ref, y_ref):
    dinv = _dinv(dp_ref[...])
    h = jnp.dot(_mix(agg_ref, x_ref, dp_ref), w_ref[...],
                preferred_element_type=jnp.float32)
    y_ref[...] = h * dinv[:, None]


def _tc_layer(agg, x, W1, degp):
    return pl.pallas_call(
        _layer_body,
        grid=(_GRID,),
        in_specs=[
            pl.BlockSpec((8, _BN, 16), lambda i: (0, i, 0)),
            pl.BlockSpec((_BN, 128), lambda i: (i, 0)),
            pl.BlockSpec((128, 128), lambda i: (0, 0)),
            pl.BlockSpec((2, _BN), lambda i: (0, i)),
        ],
        out_specs=pl.BlockSpec((_BN, 128), lambda i: (i, 0)),
        out_shape=jax.ShapeDtypeStruct((NP, 128), jnp.float32),
    )(agg, x, W1, degp)


def _final_body(agg_ref, x_ref, w_ref, wo_ref, bo_ref, dp_ref, o_ref):
    h = jnp.dot(_mix(agg_ref, x_ref, dp_ref), w_ref[...],
                preferred_element_type=jnp.float32)
    o_ref[...] = jnp.dot(h, wo_ref[...], preferred_element_type=jnp.float32) \
        + bo_ref[...]


def _tc_final(agg, x, W2, W_out_p, b_out_p, degp):
    return pl.pallas_call(
        _final_body,
        grid=(_GRID,),
        in_specs=[
            pl.BlockSpec((8, _BN, 16), lambda i: (0, i, 0)),
            pl.BlockSpec((_BN, 128), lambda i: (i, 0)),
            pl.BlockSpec((128, 128), lambda i: (0, 0)),
            pl.BlockSpec((128, 8), lambda i: (0, 0)),
            pl.BlockSpec((1, 8), lambda i: (0, 0)),
            pl.BlockSpec((2, _BN), lambda i: (0, i)),
        ],
        out_specs=pl.BlockSpec((_BN, 8), lambda i: (i, 0)),
        out_shape=jax.ShapeDtypeStruct((NP, 8), jnp.float32),
    )(agg, x, W2, W_out_p, b_out_p, degp)


def kernel(feature, edge_index, edge_type, W_in, b_in, W1, W2, W_out, b_out):
    n, _ = feature.shape
    e = edge_index.shape[1]
    pad_i = jnp.arange(E_PAD - e, dtype=jnp.int32)
    row2d = jnp.concatenate(
        [edge_index[0], pad_i % jnp.int32(n)]).reshape(EROWS_P, W)
    col2d = jnp.concatenate(
        [edge_index[1], DEAD_NODE + pad_i % jnp.int32(NP - DEAD_NODE)]
    ).reshape(EROWS_P, W)
    zeros1d = jnp.zeros((ROWS_PER_TEC,), jnp.float32)
    zeros2d = jnp.zeros((ROWS_PER_TEC, 16), jnp.float32)
    feature_p = jnp.pad(feature, ((0, NP - n), (0, 0)))
    W_out_p = jnp.pad(W_out, ((0, 0), (0, 8 - W_out.shape[1])))
    b_out_p = jnp.pad(b_out, (0, 8 - b_out.shape[0])).reshape(1, 8)

    degp = _sc_deg(col2d, zeros1d).reshape(NC, NP)
    x, y1 = _tc_prelude(feature_p, W_in, b_in, degp)
    agg1 = _sc_segsum(y1.reshape(NP * 8, 16), row2d, col2d, zeros2d)
    y2 = _tc_layer(agg1, x, W1, degp)
    agg2 = _sc_segsum(y2.reshape(NP * 8, 16), row2d, col2d, zeros2d)
    out = _tc_final(agg2, x, W2, W_out_p, b_out_p, degp)
    return out[:n, :3]
